# baseline (speedup 1.0000x reference)
import functools

import jax
import jax.numpy as jnp
from jax import lax
from jax.experimental import pallas as pl
from jax.experimental.pallas import tpu as pltpu
from jax.experimental.pallas import tpu_sc as plsc

_NN = 10000
_NP = 10240
_E = 160000
_EP = 163840
_NROW = 1280


def _vmesh():
    return plsc.VectorSubcoreMesh(core_axis_name="c", subcore_axis_name="s")


def _sc_degree(dst2, zeros640, ones128):
    @functools.partial(
        pl.kernel,
        out_type=jax.ShapeDtypeStruct((2, _NP, 128), jnp.float32),
        mesh=_vmesh(),
        scratch_types=[
            pltpu.VMEM_SHARED((_NP, 128), jnp.float32),
            pltpu.VMEM((128, 128), jnp.float32),
            pltpu.VMEM((40, 128), jnp.int32),
        ],
    )
    def k(dst2_hbm, zeros_hbm, ones_hbm, out_hbm, acc, ones_v, di):
        c = lax.axis_index("c")
        s = lax.axis_index("s")

        pltpu.sync_copy(ones_hbm, ones_v)
        pltpu.sync_copy(dst2_hbm.at[pl.ds(c * 640 + s * 40, 40)], di)
        pltpu.sync_copy(zeros_hbm, acc.at[pl.ds(s * 640, 640)])

        plsc.subcore_barrier()

        @pl.loop(0, 40)
        def _(j):
            pltpu.sync_copy(ones_v, acc.at[di.at[j]], add=True)

        plsc.subcore_barrier()
        pltpu.sync_copy(acc.at[pl.ds(s * 640, 640)],
                        out_hbm.at[c, pl.ds(s * 640, 640)])

    return k(dst2, zeros640, ones128)


def _sc_agg(g, src2, dst2, fsplit):
    nchunks = 80 if fsplit else 40

    @functools.partial(
        pl.kernel,
        out_type=jax.ShapeDtypeStruct((2, _NP, 128), jnp.float32),
        mesh=_vmesh(),
        scratch_types=[
            pltpu.VMEM_SHARED((_NP, 128), jnp.float32),
            pltpu.VMEM((128,), jnp.int32),
            pltpu.VMEM((128,), jnp.int32),
            pltpu.VMEM((128, 128), jnp.float32),
        ],
    )
    def k(g_hbm, s1_hbm, d1_hbm, out_hbm, acc, si_v, di_v, rows_v):
        c = lax.axis_index("c")
        s = lax.axis_index("s")
        gref = g_hbm.at[c] if fsplit else g_hbm

        rb = s * 640
        pltpu.sync_copy(gref.at[pl.ds(rb, 640)], acc.at[pl.ds(rb, 640)])
        plsc.subcore_barrier()

        eb = s * 10240 if fsplit else c * 81920 + s * 5120

        @pl.loop(0, nchunks)
        def _(j):
            b = eb + j * 128
            pltpu.sync_copy(s1_hbm.at[pl.ds(b, 128)], si_v)
            pltpu.sync_copy(d1_hbm.at[pl.ds(b, 128)], di_v)
            pltpu.sync_copy(gref.at[si_v], rows_v)
            pltpu.sync_copy(rows_v, acc.at[di_v], add=True)

        plsc.subcore_barrier()
        pltpu.sync_copy(acc.at[pl.ds(rb, 640)], out_hbm.at[c, pl.ds(rb, 640)])

    return k(g, src2, dst2)


def _tc_mm1(x, W1):
    def body(x_ref, w_ref, o_ref):
        o_ref[...] = jnp.dot(x_ref[...], w_ref[...],
                             preferred_element_type=jnp.float32)

    return pl.pallas_call(
        body,
        grid=(10,),
        in_specs=[
            pl.BlockSpec((1024, 256), lambda i: (i, 0)),
            pl.BlockSpec((256, 256), lambda i: (0, 0)),
        ],
        out_specs=pl.BlockSpec((1024, 256), lambda i: (i, 0)),
        out_shape=jax.ShapeDtypeStruct((_NP, 256), jnp.float32),
    )(x, W1)


def _tc_dinv(degp):
    def body(p_ref, o_ref):
        deg = jnp.max(p_ref[0], axis=1) + jnp.max(p_ref[1], axis=1) + 1.0
        o_ref[...] = lax.rsqrt(deg)

    return pl.pallas_call(
        body, out_shape=jax.ShapeDtypeStruct((_NP,), jnp.float32))(degp)


def _tc_scale1(h, dinv):
    def body(h_ref, dv_ref, o_ref):
        g = h_ref[...] * dv_ref[...][:, None]
        o_ref[0] = g[:, :128]
        o_ref[1] = g[:, 128:]

    return pl.pallas_call(
        body,
        grid=(10,),
        in_specs=[
            pl.BlockSpec((1024, 256), lambda i: (i, 0)),
            pl.BlockSpec((1024,), lambda i: (i,)),
        ],
        out_specs=pl.BlockSpec((2, 1024, 128), lambda i: (0, i, 0)),
        out_shape=jax.ShapeDtypeStruct((2, _NP, 128), jnp.float32),
    )(h, dinv)


def _tc_layer_mid(a1, dinv, b1, W2):
    def body(a_ref, dv_ref, b_ref, w_ref, o_ref):
        dv = dv_ref[...][:, None]
        t0 = a_ref[0] * dv + b_ref[...][None, :128]
        t1 = a_ref[1] * dv + b_ref[...][None, 128:]
        t = jnp.maximum(jnp.concatenate([t0, t1], axis=1), 0.0)
        o_ref[...] = jnp.dot(
            t, w_ref[...], preferred_element_type=jnp.float32) * dv

    return pl.pallas_call(
        body,
        grid=(10,),
        in_specs=[
            pl.BlockSpec((2, 1024, 128), lambda i: (0, i, 0)),
            pl.BlockSpec((1024,), lambda i: (i,)),
            pl.BlockSpec((256,), lambda i: (0,)),
            pl.BlockSpec((256, 128), lambda i: (0, 0)),
        ],
        out_specs=pl.BlockSpec((1024, 128), lambda i: (i, 0)),
        out_shape=jax.ShapeDtypeStruct((_NP, 128), jnp.float32),
    )(a1, dinv, b1, W2)


def _tc_layer2_out(a2, g2, dinv, b2):
    def body(p_ref, g_ref, dv_ref, b_ref, o_ref):
        agg = p_ref[0] + p_ref[1] - g_ref[...]
        o_ref[...] = agg * dv_ref[...][:, None] + b_ref[...][None, :]

    return pl.pallas_call(
        body,
        grid=(10,),
        in_specs=[
            pl.BlockSpec((2, 1024, 128), lambda i: (0, i, 0)),
            pl.BlockSpec((1024, 128), lambda i: (i, 0)),
            pl.BlockSpec((1024,), lambda i: (i,)),
            pl.BlockSpec((128,), lambda i: (0,)),
        ],
        out_specs=pl.BlockSpec((1024, 128), lambda i: (i, 0)),
        out_shape=jax.ShapeDtypeStruct((_NP, 128), jnp.float32),
    )(a2, g2, dinv, b2)


def kernel(x, edge_index, W1, b1, W2, b2):
    src = edge_index[0].astype(jnp.int32)
    dst = edge_index[1].astype(jnp.int32)

    npad = _EP - _E
    src1 = jnp.concatenate([src, jnp.zeros((npad,), jnp.int32)])
    dst1 = jnp.concatenate(
        [dst, _NN + (jnp.arange(npad, dtype=jnp.int32) % (_NP - _NN))])
    dst2 = dst1.reshape(_NROW, 128)
    x_p = jnp.concatenate(
        [x, jnp.zeros((_NP - _NN, x.shape[1]), jnp.float32)], axis=0)

    zeros640 = jnp.zeros((640, 128), jnp.float32)
    ones128 = jnp.ones((128, 128), jnp.float32)

    h = _tc_mm1(x_p, W1)
    degp = _sc_degree(dst2, zeros640, ones128)
    dinv = _tc_dinv(degp)
    g1 = _tc_scale1(h, dinv)
    a1 = _sc_agg(g1, src1, dst1, fsplit=True)
    g2 = _tc_layer_mid(a1, dinv, b1, W2)
    a2 = _sc_agg(g2, src1, dst1, fsplit=False)
    out = _tc_layer2_out(a2, g2, dinv, b2)
    return out[:_NN]


# speedup vs baseline: 1.8381x; 1.8381x over previous
import functools

import jax
import jax.numpy as jnp
from jax import lax
from jax.experimental import pallas as pl
from jax.experimental.pallas import tpu as pltpu
from jax.experimental.pallas import tpu_sc as plsc

_NN = 10000
_NP = 10240
_E = 160000
_EP = 163840
_NROW = 1280


def _vmesh():
    return plsc.VectorSubcoreMesh(core_axis_name="c", subcore_axis_name="s")


def _sc_degree(dst2, zeros640, ones128):
    @functools.partial(
        pl.kernel,
        out_type=jax.ShapeDtypeStruct((2, _NP, 128), jnp.float32),
        mesh=_vmesh(),
        scratch_types=[
            pltpu.VMEM_SHARED((_NP, 128), jnp.float32),
            pltpu.VMEM((128, 128), jnp.float32),
            pltpu.VMEM((40, 128), jnp.int32),
        ],
    )
    def k(dst2_hbm, zeros_hbm, ones_hbm, out_hbm, acc, ones_v, di):
        c = lax.axis_index("c")
        s = lax.axis_index("s")

        pltpu.sync_copy(ones_hbm, ones_v)
        pltpu.sync_copy(dst2_hbm.at[pl.ds(c * 640 + s * 40, 40)], di)
        pltpu.sync_copy(zeros_hbm, acc.at[pl.ds(s * 640, 640)])

        plsc.subcore_barrier()

        @pl.loop(0, 40)
        def _(j):
            pltpu.sync_copy(ones_v, acc.at[di.at[j]], add=True)

        plsc.subcore_barrier()
        pltpu.sync_copy(acc.at[pl.ds(s * 640, 640)],
                        out_hbm.at[c, pl.ds(s * 640, 640)])

    return k(dst2, zeros640, ones128)


def _sc_agg(g, src2, dst2, fsplit):
    nchunks = 80 if fsplit else 40

    @functools.partial(
        pl.kernel,
        out_type=jax.ShapeDtypeStruct((2, _NP, 128), jnp.float32),
        mesh=_vmesh(),
        scratch_types=[
            pltpu.VMEM_SHARED((_NP, 128), jnp.float32),
            pltpu.VMEM((128,), jnp.int32),
            pltpu.VMEM((128,), jnp.int32),
            pltpu.VMEM((128, 128), jnp.float32),
        ],
    )
    def k(g_hbm, s1_hbm, d1_hbm, out_hbm, acc, si_v, di_v, rows_v):
        c = lax.axis_index("c")
        s = lax.axis_index("s")
        gref = g_hbm.at[c] if fsplit else g_hbm

        rb = s * 640
        pltpu.sync_copy(gref.at[pl.ds(rb, 640)], acc.at[pl.ds(rb, 640)])
        plsc.subcore_barrier()

        eb = s * 10240 if fsplit else c * 81920 + s * 5120

        @pl.loop(0, nchunks)
        def _(j):
            b = eb + j * 128
            pltpu.sync_copy(s1_hbm.at[pl.ds(b, 128)], si_v)
            pltpu.sync_copy(d1_hbm.at[pl.ds(b, 128)], di_v)
            pltpu.sync_copy(gref.at[si_v], rows_v)
            pltpu.sync_copy(rows_v, acc.at[di_v], add=True)

        plsc.subcore_barrier()
        pltpu.sync_copy(acc.at[pl.ds(rb, 640)], out_hbm.at[c, pl.ds(rb, 640)])

    return k(g, src2, dst2)


def _tc_mm1(x, W1):
    def body(x_ref, w_ref, o_ref):
        o_ref[...] = jnp.dot(x_ref[...], w_ref[...],
                             preferred_element_type=jnp.float32)

    return pl.pallas_call(
        body,
        grid=(10,),
        in_specs=[
            pl.BlockSpec((1024, 256), lambda i: (i, 0)),
            pl.BlockSpec((256, 256), lambda i: (0, 0)),
        ],
        out_specs=pl.BlockSpec((1024, 256), lambda i: (i, 0)),
        out_shape=jax.ShapeDtypeStruct((_NP, 256), jnp.float32),
    )(x, W1)


def _tc_dinv(degp):
    def body(p_ref, o_ref):
        deg = jnp.max(p_ref[0], axis=1) + jnp.max(p_ref[1], axis=1) + 1.0
        o_ref[...] = lax.rsqrt(deg)

    return pl.pallas_call(
        body, out_shape=jax.ShapeDtypeStruct((_NP,), jnp.float32))(degp)


def _tc_scale1(h, dinv):
    def body(h_ref, dv_ref, o_ref):
        g = h_ref[...] * dv_ref[...][:, None]
        o_ref[0] = g[:, :128]
        o_ref[1] = g[:, 128:]

    return pl.pallas_call(
        body,
        grid=(10,),
        in_specs=[
            pl.BlockSpec((1024, 256), lambda i: (i, 0)),
            pl.BlockSpec((1024,), lambda i: (i,)),
        ],
        out_specs=pl.BlockSpec((2, 1024, 128), lambda i: (0, i, 0)),
        out_shape=jax.ShapeDtypeStruct((2, _NP, 128), jnp.float32),
    )(h, dinv)


def _tc_layer_mid(a1, dinv, b1, W2):
    def body(a_ref, dv_ref, b_ref, w_ref, o_ref):
        dv = dv_ref[...][:, None]
        t0 = a_ref[0] * dv + b_ref[...][None, :128]
        t1 = a_ref[1] * dv + b_ref[...][None, 128:]
        t = jnp.maximum(jnp.concatenate([t0, t1], axis=1), 0.0)
        o_ref[...] = jnp.dot(
            t, w_ref[...], preferred_element_type=jnp.float32) * dv

    return pl.pallas_call(
        body,
        grid=(10,),
        in_specs=[
            pl.BlockSpec((2, 1024, 128), lambda i: (0, i, 0)),
            pl.BlockSpec((1024,), lambda i: (i,)),
            pl.BlockSpec((256,), lambda i: (0,)),
            pl.BlockSpec((256, 128), lambda i: (0, 0)),
        ],
        out_specs=pl.BlockSpec((1024, 128), lambda i: (i, 0)),
        out_shape=jax.ShapeDtypeStruct((_NP, 128), jnp.float32),
    )(a1, dinv, b1, W2)


def _tc_layer2_out(a2, g2, dinv, b2):
    def body(p_ref, g_ref, dv_ref, b_ref, o_ref):
        agg = p_ref[0] + p_ref[1] - g_ref[...]
        o_ref[...] = agg * dv_ref[...][:, None] + b_ref[...][None, :]

    return pl.pallas_call(
        body,
        grid=(10,),
        in_specs=[
            pl.BlockSpec((2, 1024, 128), lambda i: (0, i, 0)),
            pl.BlockSpec((1024, 128), lambda i: (i, 0)),
            pl.BlockSpec((1024,), lambda i: (i,)),
            pl.BlockSpec((128,), lambda i: (0,)),
        ],
        out_specs=pl.BlockSpec((1024, 128), lambda i: (i, 0)),
        out_shape=jax.ShapeDtypeStruct((_NP, 128), jnp.float32),
    )(a2, g2, dinv, b2)


def kernel(x, edge_index, W1, b1, W2, b2):
    src = edge_index[0].astype(jnp.int32)
    dst = edge_index[1].astype(jnp.int32)

    npad = _EP - _E
    pad_iota = jnp.arange(npad, dtype=jnp.int32) % (_NP - _NN)
    src1 = jnp.concatenate([src, _NN + pad_iota])
    dst1 = jnp.concatenate([dst, _NN + pad_iota])
    dst2 = dst1.reshape(_NROW, 128)
    x_p = jnp.concatenate(
        [x, jnp.zeros((_NP - _NN, x.shape[1]), jnp.float32)], axis=0)

    zeros640 = jnp.zeros((640, 128), jnp.float32)
    ones128 = jnp.ones((128, 128), jnp.float32)

    h = _tc_mm1(x_p, W1)
    degp = _sc_degree(dst2, zeros640, ones128)
    dinv = _tc_dinv(degp)
    g1 = _tc_scale1(h, dinv)
    a1 = _sc_agg(g1, src1, dst1, fsplit=True)
    g2 = _tc_layer_mid(a1, dinv, b1, W2)
    a2 = _sc_agg(g2, src1, dst1, fsplit=False)
    out = _tc_layer2_out(a2, g2, dinv, b2)
    return out[:_NN]


# speedup vs baseline: 2.7907x; 1.5183x over previous
import functools

import jax
import jax.numpy as jnp
from jax import lax
from jax.experimental import pallas as pl
from jax.experimental.pallas import tpu as pltpu
from jax.experimental.pallas import tpu_sc as plsc

_NN = 10000
_NP = 10240
_E = 160000
_EP = 163840
_NROW = 1280


def _vmesh():
    return plsc.VectorSubcoreMesh(core_axis_name="c", subcore_axis_name="s")


def _sc_degree(dst2, zeros640, ones128):
    @functools.partial(
        pl.kernel,
        out_type=jax.ShapeDtypeStruct((2, _NP, 128), jnp.float32),
        mesh=_vmesh(),
        scratch_types=[
            pltpu.VMEM_SHARED((_NP, 128), jnp.float32),
            pltpu.VMEM((128, 128), jnp.float32),
            pltpu.VMEM((40, 128), jnp.int32),
        ],
    )
    def k(dst2_hbm, zeros_hbm, ones_hbm, out_hbm, acc, ones_v, di):
        c = lax.axis_index("c")
        s = lax.axis_index("s")

        pltpu.sync_copy(ones_hbm, ones_v)
        pltpu.sync_copy(dst2_hbm.at[pl.ds(c * 640 + s * 40, 40)], di)
        pltpu.sync_copy(zeros_hbm, acc.at[pl.ds(s * 640, 640)])

        plsc.subcore_barrier()

        @pl.loop(0, 40)
        def _(j):
            pltpu.sync_copy(ones_v, acc.at[di.at[j]], add=True)

        plsc.subcore_barrier()
        pltpu.sync_copy(acc.at[pl.ds(s * 640, 640)],
                        out_hbm.at[c, pl.ds(s * 640, 640)])

    return k(dst2, zeros640, ones128)


def _sc_agg(g, src2, dst2, fsplit):
    npass = 2 if fsplit else 1

    @functools.partial(
        pl.kernel,
        out_type=jax.ShapeDtypeStruct((2, _NP, 128), jnp.float32),
        mesh=_vmesh(),
        scratch_types=[
            pltpu.VMEM_SHARED((_NP, 128), jnp.float32),
            pltpu.VMEM((40, 128), jnp.int32),
            pltpu.VMEM((40, 128), jnp.int32),
            pltpu.VMEM((128, 128), jnp.float32),
            pltpu.VMEM((128, 128), jnp.float32),
            pltpu.SemaphoreType.DMA,
            pltpu.SemaphoreType.DMA,
        ],
    )
    def k(g_hbm, s2_hbm, d2_hbm, out_hbm, acc, si, di, r0, r1, gs0, gs1):
        c = lax.axis_index("c")
        s = lax.axis_index("s")
        gref = g_hbm.at[c] if fsplit else g_hbm

        rb = s * 640
        pltpu.sync_copy(gref.at[pl.ds(rb, 640)], acc.at[pl.ds(rb, 640)])
        plsc.subcore_barrier()

        for p in range(npass):
            irow = (s * 80 + p * 40) if fsplit else c * 640 + s * 40
            pltpu.sync_copy(s2_hbm.at[pl.ds(irow, 40)], si)
            pltpu.sync_copy(d2_hbm.at[pl.ds(irow, 40)], di)

            pltpu.async_copy(gref.at[si.at[0]], r0, gs0).wait()

            @pl.loop(0, 19)
            def _(jj):
                j = jj * 2
                h1 = pltpu.async_copy(gref.at[si.at[j + 1]], r1, gs1)
                pltpu.sync_copy(r0, acc.at[di.at[j]], add=True)
                h1.wait()
                h2 = pltpu.async_copy(gref.at[si.at[j + 2]], r0, gs0)
                pltpu.sync_copy(r1, acc.at[di.at[j + 1]], add=True)
                h2.wait()

            h1 = pltpu.async_copy(gref.at[si.at[39]], r1, gs1)
            pltpu.sync_copy(r0, acc.at[di.at[38]], add=True)
            h1.wait()
            pltpu.sync_copy(r1, acc.at[di.at[39]], add=True)

        plsc.subcore_barrier()
        pltpu.sync_copy(acc.at[pl.ds(rb, 640)], out_hbm.at[c, pl.ds(rb, 640)])

    return k(g, src2, dst2)


def _tc_mm1(x, W1):
    def body(x_ref, w_ref, o_ref):
        o_ref[...] = jnp.dot(x_ref[...], w_ref[...],
                             preferred_element_type=jnp.float32)

    return pl.pallas_call(
        body,
        grid=(10,),
        in_specs=[
            pl.BlockSpec((1024, 256), lambda i: (i, 0)),
            pl.BlockSpec((256, 256), lambda i: (0, 0)),
        ],
        out_specs=pl.BlockSpec((1024, 256), lambda i: (i, 0)),
        out_shape=jax.ShapeDtypeStruct((_NP, 256), jnp.float32),
    )(x, W1)


def _tc_dinv(degp):
    def body(p_ref, o_ref):
        deg = jnp.max(p_ref[0], axis=1) + jnp.max(p_ref[1], axis=1) + 1.0
        o_ref[...] = lax.rsqrt(deg)

    return pl.pallas_call(
        body, out_shape=jax.ShapeDtypeStruct((_NP,), jnp.float32))(degp)


def _tc_scale1(h, dinv):
    def body(h_ref, dv_ref, o_ref):
        g = h_ref[...] * dv_ref[...][:, None]
        o_ref[0] = g[:, :128]
        o_ref[1] = g[:, 128:]

    return pl.pallas_call(
        body,
        grid=(10,),
        in_specs=[
            pl.BlockSpec((1024, 256), lambda i: (i, 0)),
            pl.BlockSpec((1024,), lambda i: (i,)),
        ],
        out_specs=pl.BlockSpec((2, 1024, 128), lambda i: (0, i, 0)),
        out_shape=jax.ShapeDtypeStruct((2, _NP, 128), jnp.float32),
    )(h, dinv)


def _tc_layer_mid(a1, dinv, b1, W2):
    def body(a_ref, dv_ref, b_ref, w_ref, o_ref):
        dv = dv_ref[...][:, None]
        t0 = a_ref[0] * dv + b_ref[...][None, :128]
        t1 = a_ref[1] * dv + b_ref[...][None, 128:]
        t = jnp.maximum(jnp.concatenate([t0, t1], axis=1), 0.0)
        o_ref[...] = jnp.dot(
            t, w_ref[...], preferred_element_type=jnp.float32) * dv

    return pl.pallas_call(
        body,
        grid=(10,),
        in_specs=[
            pl.BlockSpec((2, 1024, 128), lambda i: (0, i, 0)),
            pl.BlockSpec((1024,), lambda i: (i,)),
            pl.BlockSpec((256,), lambda i: (0,)),
            pl.BlockSpec((256, 128), lambda i: (0, 0)),
        ],
        out_specs=pl.BlockSpec((1024, 128), lambda i: (i, 0)),
        out_shape=jax.ShapeDtypeStruct((_NP, 128), jnp.float32),
    )(a1, dinv, b1, W2)


def _tc_layer2_out(a2, g2, dinv, b2):
    def body(p_ref, g_ref, dv_ref, b_ref, o_ref):
        agg = p_ref[0] + p_ref[1] - g_ref[...]
        o_ref[...] = agg * dv_ref[...][:, None] + b_ref[...][None, :]

    return pl.pallas_call(
        body,
        grid=(10,),
        in_specs=[
            pl.BlockSpec((2, 1024, 128), lambda i: (0, i, 0)),
            pl.BlockSpec((1024, 128), lambda i: (i, 0)),
            pl.BlockSpec((1024,), lambda i: (i,)),
            pl.BlockSpec((128,), lambda i: (0,)),
        ],
        out_specs=pl.BlockSpec((1024, 128), lambda i: (i, 0)),
        out_shape=jax.ShapeDtypeStruct((_NP, 128), jnp.float32),
    )(a2, g2, dinv, b2)


def kernel(x, edge_index, W1, b1, W2, b2):
    src = edge_index[0].astype(jnp.int32)
    dst = edge_index[1].astype(jnp.int32)

    npad = _EP - _E
    pad_iota = jnp.arange(npad, dtype=jnp.int32) % (_NP - _NN)
    src1 = jnp.concatenate([src, _NN + pad_iota])
    dst1 = jnp.concatenate([dst, _NN + pad_iota])
    dst2 = dst1.reshape(_NROW, 128)
    x_p = jnp.concatenate(
        [x, jnp.zeros((_NP - _NN, x.shape[1]), jnp.float32)], axis=0)

    zeros640 = jnp.zeros((640, 128), jnp.float32)
    ones128 = jnp.ones((128, 128), jnp.float32)

    h = _tc_mm1(x_p, W1)
    degp = _sc_degree(dst2, zeros640, ones128)
    dinv = _tc_dinv(degp)
    g1 = _tc_scale1(h, dinv)
    src2 = src1.reshape(_NROW, 128)
    a1 = _sc_agg(g1, src2, dst2, fsplit=True)
    g2 = _tc_layer_mid(a1, dinv, b1, W2)
    a2 = _sc_agg(g2, src2, dst2, fsplit=False)
    out = _tc_layer2_out(a2, g2, dinv, b2)
    return out[:_NN]


# speedup vs baseline: 2.8903x; 1.0357x over previous
import functools

import jax
import jax.numpy as jnp
from jax import lax
from jax.experimental import pallas as pl
from jax.experimental.pallas import tpu as pltpu
from jax.experimental.pallas import tpu_sc as plsc

_NN = 10000
_NP = 10240
_E = 160000
_EP = 163840
_NROW = 1280


def _vmesh():
    return plsc.VectorSubcoreMesh(core_axis_name="c", subcore_axis_name="s")


def _sc_degree(dst2, zeros640, ones128):
    @functools.partial(
        pl.kernel,
        out_type=jax.ShapeDtypeStruct((2, _NP, 128), jnp.float32),
        mesh=_vmesh(),
        scratch_types=[
            pltpu.VMEM_SHARED((_NP, 128), jnp.float32),
            pltpu.VMEM((128, 128), jnp.float32),
            pltpu.VMEM((40, 128), jnp.int32),
        ],
    )
    def k(dst2_hbm, zeros_hbm, ones_hbm, out_hbm, acc, ones_v, di):
        c = lax.axis_index("c")
        s = lax.axis_index("s")

        pltpu.sync_copy(ones_hbm, ones_v)
        pltpu.sync_copy(dst2_hbm.at[pl.ds(c * 640 + s * 40, 40)], di)
        pltpu.sync_copy(zeros_hbm, acc.at[pl.ds(s * 640, 640)])

        plsc.subcore_barrier()

        @pl.loop(0, 40)
        def _(j):
            pltpu.sync_copy(ones_v, acc.at[di.at[j]], add=True)

        plsc.subcore_barrier()
        pltpu.sync_copy(acc.at[pl.ds(s * 640, 640)],
                        out_hbm.at[c, pl.ds(s * 640, 640)])

    return k(dst2, zeros640, ones128)


def _sc_agg(g, src2, dst2, fsplit):
    npass = 2 if fsplit else 1

    @functools.partial(
        pl.kernel,
        out_type=jax.ShapeDtypeStruct((2, _NP, 128), jnp.float32),
        mesh=_vmesh(),
        scratch_types=[
            pltpu.VMEM_SHARED((_NP, 128), jnp.float32),
            pltpu.VMEM((40, 128), jnp.int32),
            pltpu.VMEM((40, 128), jnp.int32),
            pltpu.VMEM((128, 128), jnp.float32),
            pltpu.VMEM((128, 128), jnp.float32),
            pltpu.SemaphoreType.DMA,
            pltpu.SemaphoreType.DMA,
        ],
    )
    def k(g_hbm, s2_hbm, d2_hbm, out_hbm, acc, si, di, r0, r1, gs0, gs1):
        c = lax.axis_index("c")
        s = lax.axis_index("s")
        gref = g_hbm.at[c] if fsplit else g_hbm

        rb = s * 640
        pltpu.sync_copy(gref.at[pl.ds(rb, 640)], acc.at[pl.ds(rb, 640)])
        plsc.subcore_barrier()

        for p in range(npass):
            irow = (s * 80 + p * 40) if fsplit else c * 640 + s * 40
            pltpu.sync_copy(s2_hbm.at[pl.ds(irow, 40)], si)
            pltpu.sync_copy(d2_hbm.at[pl.ds(irow, 40)], di)

            pltpu.async_copy(gref.at[si.at[0]], r0, gs0).wait()

            @pl.loop(0, 19)
            def _(jj):
                j = jj * 2
                h1 = pltpu.async_copy(gref.at[si.at[j + 1]], r1, gs1)
                pltpu.sync_copy(r0, acc.at[di.at[j]], add=True)
                h1.wait()
                h2 = pltpu.async_copy(gref.at[si.at[j + 2]], r0, gs0)
                pltpu.sync_copy(r1, acc.at[di.at[j + 1]], add=True)
                h2.wait()

            h1 = pltpu.async_copy(gref.at[si.at[39]], r1, gs1)
            pltpu.sync_copy(r0, acc.at[di.at[38]], add=True)
            h1.wait()
            pltpu.sync_copy(r1, acc.at[di.at[39]], add=True)

        plsc.subcore_barrier()
        pltpu.sync_copy(acc.at[pl.ds(rb, 640)], out_hbm.at[c, pl.ds(rb, 640)])

    return k(g, src2, dst2)


def _tc_mm1(x, W1):
    def body(x_ref, w_ref, o_ref):
        o_ref[...] = jnp.dot(x_ref[...], w_ref[...],
                             preferred_element_type=jnp.float32)

    return pl.pallas_call(
        body,
        grid=(10,),
        in_specs=[
            pl.BlockSpec((1024, 256), lambda i: (i, 0)),
            pl.BlockSpec((256, 256), lambda i: (0, 0)),
        ],
        out_specs=pl.BlockSpec((1024, 256), lambda i: (i, 0)),
        out_shape=jax.ShapeDtypeStruct((_NP, 256), jnp.float32),
    )(x, W1)


def _tc_scale1(h, degp):
    def body(h_ref, p_ref, o_ref, dv_ref):
        deg = jnp.max(p_ref[0], axis=1) + jnp.max(p_ref[1], axis=1) + 1.0
        dv = lax.rsqrt(deg)
        dv_ref[...] = dv
        g = h_ref[...] * dv[:, None]
        o_ref[0] = g[:, :128]
        o_ref[1] = g[:, 128:]

    return pl.pallas_call(
        body,
        grid=(10,),
        in_specs=[
            pl.BlockSpec((1024, 256), lambda i: (i, 0)),
            pl.BlockSpec((2, 1024, 128), lambda i: (0, i, 0)),
        ],
        out_specs=[
            pl.BlockSpec((2, 1024, 128), lambda i: (0, i, 0)),
            pl.BlockSpec((1024,), lambda i: (i,)),
        ],
        out_shape=[
            jax.ShapeDtypeStruct((2, _NP, 128), jnp.float32),
            jax.ShapeDtypeStruct((_NP,), jnp.float32),
        ],
    )(h, degp)


def _tc_layer_mid(a1, dinv, b1, W2):
    def body(a_ref, dv_ref, b_ref, w_ref, o_ref):
        dv = dv_ref[...][:, None]
        t0 = a_ref[0] * dv + b_ref[...][None, :128]
        t1 = a_ref[1] * dv + b_ref[...][None, 128:]
        t = jnp.maximum(jnp.concatenate([t0, t1], axis=1), 0.0)
        o_ref[...] = jnp.dot(
            t, w_ref[...], preferred_element_type=jnp.float32) * dv

    return pl.pallas_call(
        body,
        grid=(10,),
        in_specs=[
            pl.BlockSpec((2, 1024, 128), lambda i: (0, i, 0)),
            pl.BlockSpec((1024,), lambda i: (i,)),
            pl.BlockSpec((256,), lambda i: (0,)),
            pl.BlockSpec((256, 128), lambda i: (0, 0)),
        ],
        out_specs=pl.BlockSpec((1024, 128), lambda i: (i, 0)),
        out_shape=jax.ShapeDtypeStruct((_NP, 128), jnp.float32),
    )(a1, dinv, b1, W2)


def _tc_layer2_out(a2, g2, dinv, b2):
    def body(p_ref, g_ref, dv_ref, b_ref, o_ref):
        agg = p_ref[0] + p_ref[1] - g_ref[...]
        o_ref[...] = agg * dv_ref[...][:, None] + b_ref[...][None, :]

    return pl.pallas_call(
        body,
        grid=(10,),
        in_specs=[
            pl.BlockSpec((2, 1024, 128), lambda i: (0, i, 0)),
            pl.BlockSpec((1024, 128), lambda i: (i, 0)),
            pl.BlockSpec((1024,), lambda i: (i,)),
            pl.BlockSpec((128,), lambda i: (0,)),
        ],
        out_specs=pl.BlockSpec((1024, 128), lambda i: (i, 0)),
        out_shape=jax.ShapeDtypeStruct((_NN, 128), jnp.float32),
    )(a2, g2, dinv, b2)


def kernel(x, edge_index, W1, b1, W2, b2):
    src = edge_index[0].astype(jnp.int32)
    dst = edge_index[1].astype(jnp.int32)

    npad = _EP - _E
    pad_iota = jnp.arange(npad, dtype=jnp.int32) % (_NP - _NN)
    src1 = jnp.concatenate([src, _NN + pad_iota])
    dst1 = jnp.concatenate([dst, _NN + pad_iota])
    dst2 = dst1.reshape(_NROW, 128)
    x_p = jnp.concatenate(
        [x, jnp.zeros((_NP - _NN, x.shape[1]), jnp.float32)], axis=0)

    zeros640 = jnp.zeros((640, 128), jnp.float32)
    ones128 = jnp.ones((128, 128), jnp.float32)

    h = _tc_mm1(x_p, W1)
    degp = _sc_degree(dst2, zeros640, ones128)
    g1, dinv = _tc_scale1(h, degp)
    src2 = src1.reshape(_NROW, 128)
    a1 = _sc_agg(g1, src2, dst2, fsplit=True)
    g2 = _tc_layer_mid(a1, dinv, b1, W2)
    a2 = _sc_agg(g2, src2, dst2, fsplit=False)
    return _tc_layer2_out(a2, g2, dinv, b2)


# speedup vs baseline: 3.2189x; 1.1137x over previous
import dataclasses
import functools

import jax
import jax.numpy as jnp
from jax import lax
from jax.experimental import pallas as pl
from jax.experimental.pallas import tpu as pltpu
from jax.experimental.pallas import tpu_sc as plsc

_NN = 10000
_NP = 10240
_E = 160000
_EP = 163840
_NROW = 1280


def _vmesh():
    return plsc.VectorSubcoreMesh(core_axis_name="c", subcore_axis_name="s")


_degree_cp = pltpu.CompilerParams()
if "needs_layout_passes" in pltpu.CompilerParams.__dataclass_fields__:
    _degree_cp = dataclasses.replace(_degree_cp, needs_layout_passes=False)


def _sc_degree(dst2, zeros_np):
    @functools.partial(
        pl.kernel,
        out_type=jax.ShapeDtypeStruct((2, 16, _NP), jnp.float32),
        mesh=_vmesh(),
        compiler_params=_degree_cp,
        scratch_types=[
            pltpu.VMEM((_NP,), jnp.float32),
            pltpu.VMEM((40, 128), jnp.int32),
        ],
    )
    def k(dst2_hbm, zeros_hbm, out_hbm, cnt, di):
        c = lax.axis_index("c")
        s = lax.axis_index("s")
        pltpu.sync_copy(zeros_hbm, cnt)
        pltpu.sync_copy(dst2_hbm.at[pl.ds(c * 640 + s * 40, 40)], di)
        ones16 = jnp.ones((16,), jnp.float32)

        @pl.loop(0, 40)
        def _(j):
            @pl.loop(0, 8)
            def _(kk):
                idx16 = di[j, pl.ds(kk * 16, 16)]
                plsc.addupdate_scatter(cnt, [idx16], ones16)

        plsc.subcore_barrier()
        pltpu.sync_copy(cnt, out_hbm.at[c, s])

    return k(dst2, zeros_np)


def _sc_agg(g, src2, dst2, fsplit):
    npass = 2 if fsplit else 1

    @functools.partial(
        pl.kernel,
        out_type=jax.ShapeDtypeStruct((2, _NP, 128), jnp.float32),
        mesh=_vmesh(),
        scratch_types=[
            pltpu.VMEM_SHARED((_NP, 128), jnp.float32),
            pltpu.VMEM((40, 128), jnp.int32),
            pltpu.VMEM((40, 128), jnp.int32),
            pltpu.VMEM((128, 128), jnp.float32),
            pltpu.VMEM((128, 128), jnp.float32),
            pltpu.SemaphoreType.DMA,
            pltpu.SemaphoreType.DMA,
        ],
    )
    def k(g_hbm, s2_hbm, d2_hbm, out_hbm, acc, si, di, r0, r1, gs0, gs1):
        c = lax.axis_index("c")
        s = lax.axis_index("s")
        gref = g_hbm.at[c] if fsplit else g_hbm

        rb = s * 640
        pltpu.sync_copy(gref.at[pl.ds(rb, 640)], acc.at[pl.ds(rb, 640)])
        plsc.subcore_barrier()

        for p in range(npass):
            irow = (s * 80 + p * 40) if fsplit else c * 640 + s * 40
            pltpu.sync_copy(s2_hbm.at[pl.ds(irow, 40)], si)
            pltpu.sync_copy(d2_hbm.at[pl.ds(irow, 40)], di)

            pltpu.async_copy(gref.at[si.at[0]], r0, gs0).wait()

            @pl.loop(0, 19)
            def _(jj):
                j = jj * 2
                h1 = pltpu.async_copy(gref.at[si.at[j + 1]], r1, gs1)
                pltpu.sync_copy(r0, acc.at[di.at[j]], add=True)
                h1.wait()
                h2 = pltpu.async_copy(gref.at[si.at[j + 2]], r0, gs0)
                pltpu.sync_copy(r1, acc.at[di.at[j + 1]], add=True)
                h2.wait()

            h1 = pltpu.async_copy(gref.at[si.at[39]], r1, gs1)
            pltpu.sync_copy(r0, acc.at[di.at[38]], add=True)
            h1.wait()
            pltpu.sync_copy(r1, acc.at[di.at[39]], add=True)

        plsc.subcore_barrier()
        pltpu.sync_copy(acc.at[pl.ds(rb, 640)], out_hbm.at[c, pl.ds(rb, 640)])

    return k(g, src2, dst2)


def _tc_mm1(x, W1):
    def body(x_ref, w_ref, o_ref):
        o_ref[...] = jnp.dot(x_ref[...], w_ref[...],
                             preferred_element_type=jnp.float32)

    return pl.pallas_call(
        body,
        grid=(10,),
        in_specs=[
            pl.BlockSpec((1024, 256), lambda i: (i, 0)),
            pl.BlockSpec((256, 256), lambda i: (0, 0)),
        ],
        out_specs=pl.BlockSpec((1024, 256), lambda i: (i, 0)),
        out_shape=jax.ShapeDtypeStruct((_NP, 256), jnp.float32),
    )(x, W1)


def _tc_scale1(h, degp):
    def body(h_ref, p_ref, o_ref, dv_ref):
        deg = jnp.sum(p_ref[0] + p_ref[1], axis=0) + 1.0
        dv = lax.rsqrt(deg)
        dv_ref[...] = dv
        g = h_ref[...] * dv[:, None]
        o_ref[0] = g[:, :128]
        o_ref[1] = g[:, 128:]

    return pl.pallas_call(
        body,
        grid=(10,),
        in_specs=[
            pl.BlockSpec((1024, 256), lambda i: (i, 0)),
            pl.BlockSpec((2, 16, 1024), lambda i: (0, 0, i)),
        ],
        out_specs=[
            pl.BlockSpec((2, 1024, 128), lambda i: (0, i, 0)),
            pl.BlockSpec((1024,), lambda i: (i,)),
        ],
        out_shape=[
            jax.ShapeDtypeStruct((2, _NP, 128), jnp.float32),
            jax.ShapeDtypeStruct((_NP,), jnp.float32),
        ],
    )(h, degp)


def _tc_layer_mid(a1, dinv, b1, W2):
    def body(a_ref, dv_ref, b_ref, w_ref, o_ref):
        dv = dv_ref[...][:, None]
        t0 = a_ref[0] * dv + b_ref[...][None, :128]
        t1 = a_ref[1] * dv + b_ref[...][None, 128:]
        t = jnp.maximum(jnp.concatenate([t0, t1], axis=1), 0.0)
        o_ref[...] = jnp.dot(
            t, w_ref[...], preferred_element_type=jnp.float32) * dv

    return pl.pallas_call(
        body,
        grid=(10,),
        in_specs=[
            pl.BlockSpec((2, 1024, 128), lambda i: (0, i, 0)),
            pl.BlockSpec((1024,), lambda i: (i,)),
            pl.BlockSpec((256,), lambda i: (0,)),
            pl.BlockSpec((256, 128), lambda i: (0, 0)),
        ],
        out_specs=pl.BlockSpec((1024, 128), lambda i: (i, 0)),
        out_shape=jax.ShapeDtypeStruct((_NP, 128), jnp.float32),
    )(a1, dinv, b1, W2)


def _tc_layer2_out(a2, g2, dinv, b2):
    def body(p_ref, g_ref, dv_ref, b_ref, o_ref):
        agg = p_ref[0] + p_ref[1] - g_ref[...]
        o_ref[...] = agg * dv_ref[...][:, None] + b_ref[...][None, :]

    return pl.pallas_call(
        body,
        grid=(10,),
        in_specs=[
            pl.BlockSpec((2, 1024, 128), lambda i: (0, i, 0)),
            pl.BlockSpec((1024, 128), lambda i: (i, 0)),
            pl.BlockSpec((1024,), lambda i: (i,)),
            pl.BlockSpec((128,), lambda i: (0,)),
        ],
        out_specs=pl.BlockSpec((1024, 128), lambda i: (i, 0)),
        out_shape=jax.ShapeDtypeStruct((_NN, 128), jnp.float32),
    )(a2, g2, dinv, b2)


def kernel(x, edge_index, W1, b1, W2, b2):
    src = edge_index[0].astype(jnp.int32)
    dst = edge_index[1].astype(jnp.int32)

    npad = _EP - _E
    pad_iota = jnp.arange(npad, dtype=jnp.int32) % (_NP - _NN)
    src1 = jnp.concatenate([src, _NN + pad_iota])
    dst1 = jnp.concatenate([dst, _NN + pad_iota])
    dst2 = dst1.reshape(_NROW, 128)
    x_p = jnp.concatenate(
        [x, jnp.zeros((_NP - _NN, x.shape[1]), jnp.float32)], axis=0)

    h = _tc_mm1(x_p, W1)
    degp = _sc_degree(dst2, jnp.zeros((_NP,), jnp.float32))
    g1, dinv = _tc_scale1(h, degp)
    src2 = src1.reshape(_NROW, 128)
    a1 = _sc_agg(g1, src2, dst2, fsplit=True)
    g2 = _tc_layer_mid(a1, dinv, b1, W2)
    a2 = _sc_agg(g2, src2, dst2, fsplit=False)
    return _tc_layer2_out(a2, g2, dinv, b2)
